# Initial kernel scaffold; baseline (speedup 1.0000x reference)
import jax
import jax.numpy as jnp
from jax.experimental import pallas as pl


def kernel(x, edge_index, edge_weight, Win, bin_, convW, convB, lnG, lnB):
    raise NotImplementedError("write your pallas kernel here")


# speedup vs baseline: 8.5525x; 8.5525x over previous
import functools

import jax
import jax.numpy as jnp
from jax import lax
from jax.experimental import pallas as pl
from jax.experimental.pallas import tpu as pltpu
from jax.experimental.pallas import tpu_sc as plsc

NC = 2
NS = 16
LANES = 16
NW = NC * NS

BK = 128


def _mesh():
    return plsc.VectorSubcoreMesh(
        core_axis_name="c", subcore_axis_name="s", num_cores=NC, num_subcores=NS
    )


def _tc_linear(x, wT, b, relu):
    n, d = x.shape
    blk = 1000
    grid = n // blk

    def body(x_ref, w_ref, b_ref, o_ref):
        acc = jnp.dot(x_ref[...], w_ref[...], preferred_element_type=jnp.float32)
        if b_ref is not None:
            acc = acc + b_ref[...]
        if relu:
            acc = jnp.maximum(acc, 0.0)
        o_ref[...] = acc

    if b is None:
        def body2(x_ref, w_ref, o_ref):
            return body(x_ref, w_ref, None, o_ref)
        in_specs = [
            pl.BlockSpec((blk, d), lambda i: (i, 0)),
            pl.BlockSpec((d, d), lambda i: (0, 0)),
        ]
        return pl.pallas_call(
            body2,
            grid=(grid,),
            in_specs=in_specs,
            out_specs=pl.BlockSpec((blk, d), lambda i: (i, 0)),
            out_shape=jax.ShapeDtypeStruct((n, d), jnp.float32),
        )(x, wT)

    in_specs = [
        pl.BlockSpec((blk, d), lambda i: (i, 0)),
        pl.BlockSpec((d, d), lambda i: (0, 0)),
        pl.BlockSpec((1, d), lambda i: (0, 0)),
    ]
    return pl.pallas_call(
        body,
        grid=(grid,),
        in_specs=in_specs,
        out_specs=pl.BlockSpec((blk, d), lambda i: (i, 0)),
        out_shape=jax.ShapeDtypeStruct((n, d), jnp.float32),
    )(x, wT, b.reshape(1, d))


def _tc_dinv(degp, n):
    def body(p_ref, dinv_ref, dinv2_ref):
        deg = jnp.sum(p_ref[...], axis=0, keepdims=True) + 1.0
        dinv = jnp.where(deg > 0, lax.rsqrt(deg), 0.0)
        dinv_ref[...] = dinv
        dinv2_ref[...] = dinv * dinv

    return pl.pallas_call(
        body,
        out_shape=(
            jax.ShapeDtypeStruct((1, n), jnp.float32),
            jax.ShapeDtypeStruct((1, n), jnp.float32),
        ),
    )(degp)


def _tc_epilogue(scat, g, dinv2, b, gam, bet, res):
    n, d = g.shape
    blk = 1000
    grid = n // blk

    def body(s_ref, g_ref, d2_ref, b_ref, gam_ref, bet_ref, r_ref, o_ref):
        t = s_ref[0] + s_ref[1] + d2_ref[...] * g_ref[...] + b_ref[...]
        m = jnp.mean(t, axis=-1, keepdims=True)
        tc = t - m
        v = jnp.mean(tc * tc, axis=-1, keepdims=True)
        y = tc * lax.rsqrt(v + 1e-5) * gam_ref[...] + bet_ref[...]
        o_ref[...] = jnp.maximum(y, 0.0) + r_ref[...]

    return pl.pallas_call(
        body,
        grid=(grid,),
        in_specs=[
            pl.BlockSpec((2, blk, d), lambda i: (0, i, 0)),
            pl.BlockSpec((blk, d), lambda i: (i, 0)),
            pl.BlockSpec((blk, 1), lambda i: (i, 0)),
            pl.BlockSpec((1, d), lambda i: (0, 0)),
            pl.BlockSpec((1, d), lambda i: (0, 0)),
            pl.BlockSpec((1, d), lambda i: (0, 0)),
            pl.BlockSpec((blk, d), lambda i: (i, 0)),
        ],
        out_specs=pl.BlockSpec((blk, d), lambda i: (i, 0)),
        out_shape=jax.ShapeDtypeStruct((n, d), jnp.float32),
    )(scat, g, dinv2, b.reshape(1, d), gam.reshape(1, d), bet.reshape(1, d), res)


def _sc_deg(dst_w, ew_w, n, nb):
    def body(dst_hbm, ew_hbm, out_hbm, dst_v, ew_v, deg_v, sem):
        c = lax.axis_index("c")
        s = lax.axis_index("s")
        wid = c * NS + s
        pltpu.sync_copy(dst_hbm.at[wid], dst_v)
        pltpu.sync_copy(ew_hbm.at[wid], ew_v)
        zv = jnp.zeros((LANES,), jnp.float32)

        def zbody(i, _):
            deg_v[pl.ds(i * LANES, LANES)] = zv
            return 0

        lax.fori_loop(0, n // LANES, zbody, 0, unroll=8)

        def ebody(bidx, _):
            for k in range(BK // LANES):
                dvec = dst_v[bidx, pl.ds(k * LANES, LANES)]
                wvec = ew_v[bidx, pl.ds(k * LANES, LANES)]
                plsc.addupdate_scatter(deg_v, [dvec], wvec)
            return 0

        lax.fori_loop(0, nb, ebody, 0)
        pltpu.sync_copy(deg_v, out_hbm.at[wid])

    f = pl.kernel(
        body,
        out_type=jax.ShapeDtypeStruct((NW, n), jnp.float32),
        mesh=_mesh(),
        compiler_params=pltpu.CompilerParams(needs_layout_passes=False),
        scratch_types=[
            pltpu.VMEM((nb, BK), jnp.int32),
            pltpu.VMEM((nb, BK), jnp.float32),
            pltpu.VMEM((n,), jnp.float32),
            pltpu.SemaphoreType.DMA,
        ],
    )
    return f(dst_w, ew_w)


def _sc_norm(src_w, dst_w, ew_w, dinv, n, nb):
    def body(src_hbm, dst_hbm, ew_hbm, dinv_hbm, out_hbm,
             src_v, dst_v, ew_v, nrm_v, dinv_v, sem):
        c = lax.axis_index("c")
        s = lax.axis_index("s")
        wid = c * NS + s
        pltpu.sync_copy(dinv_hbm, dinv_v)
        pltpu.sync_copy(src_hbm.at[wid], src_v)
        pltpu.sync_copy(dst_hbm.at[wid], dst_v)
        pltpu.sync_copy(ew_hbm.at[wid], ew_v)

        def ebody(bidx, _):
            for k in range(BK // LANES):
                sl = pl.ds(k * LANES, LANES)
                svec = src_v[bidx, sl]
                dvec = dst_v[bidx, sl]
                wvec = ew_v[bidx, sl]
                a = plsc.load_gather(dinv_v, [svec])
                bb = plsc.load_gather(dinv_v, [dvec])
                nrm_v[bidx, sl] = a * wvec * bb
            return 0

        lax.fori_loop(0, nb, ebody, 0)
        pltpu.sync_copy(nrm_v, out_hbm.at[wid])

    f = pl.kernel(
        body,
        out_type=jax.ShapeDtypeStruct((NW, nb, BK), jnp.float32),
        mesh=_mesh(),
        compiler_params=pltpu.CompilerParams(needs_layout_passes=False),
        scratch_types=[
            pltpu.VMEM((nb, BK), jnp.int32),
            pltpu.VMEM((nb, BK), jnp.int32),
            pltpu.VMEM((nb, BK), jnp.float32),
            pltpu.VMEM((nb, BK), jnp.float32),
            pltpu.VMEM((n,), jnp.float32),
            pltpu.SemaphoreType.DMA,
        ],
    )
    return f(src_w, dst_w, ew_w, dinv)


def _sc_agg(g, src_w, dst_w, nrm_w, zeros, n, d, nb):
    npad = zeros.shape[0]
    rpt = npad // NS

    def body(g_hbm, src_hbm, dst_hbm, nrm_hbm, z_hbm, out_hbm,
             src_v, dst_v, nrm_v, rows_v, acc, sem):
        c = lax.axis_index("c")
        s = lax.axis_index("s")
        wid = c * NS + s
        pltpu.sync_copy(z_hbm.at[pl.ds(s * rpt, rpt)], acc.at[pl.ds(s * rpt, rpt)])
        pltpu.sync_copy(src_hbm.at[wid], src_v)
        pltpu.sync_copy(dst_hbm.at[wid], dst_v)
        pltpu.sync_copy(nrm_hbm.at[wid], nrm_v)
        plsc.subcore_barrier()

        def ebody(bidx, _):
            pltpu.async_copy(g_hbm.at[src_v.at[bidx]], rows_v, sem).wait()

            def rbody(q, _):
                nv = nrm_v[bidx, pl.ds(q * LANES, LANES)]
                for r2 in range(LANES):
                    sc = nv[r2]
                    r = q * LANES + r2
                    for j in range(d // LANES):
                        sl = pl.ds(j * LANES, LANES)
                        rows_v[r, sl] = rows_v[r, sl] * sc
                return 0

            lax.fori_loop(0, BK // LANES, rbody, 0)
            pltpu.sync_copy(rows_v, acc.at[dst_v.at[bidx]], add=True)
            return 0

        lax.fori_loop(0, nb, ebody, 0)
        plsc.subcore_barrier()
        pltpu.sync_copy(acc.at[pl.ds(s * rpt, rpt)],
                        out_hbm.at[c, pl.ds(s * rpt, rpt)])

    f = pl.kernel(
        body,
        out_type=jax.ShapeDtypeStruct((NC, npad, d), jnp.float32),
        mesh=_mesh(),
        compiler_params=pltpu.CompilerParams(needs_layout_passes=False),
        scratch_types=[
            pltpu.VMEM((nb, BK), jnp.int32),
            pltpu.VMEM((nb, BK), jnp.int32),
            pltpu.VMEM((nb, BK), jnp.float32),
            pltpu.VMEM((BK, d), jnp.float32),
            pltpu.VMEM_SHARED((npad, d), jnp.float32),
            pltpu.SemaphoreType.DMA,
        ],
    )
    return f(g, src_w, dst_w, nrm_w, zeros)


def kernel(x, edge_index, edge_weight, Win, bin_, convW, convB, lnG, lnB):
    n, d = x.shape
    e = edge_weight.shape[0]
    nlayer = convW.shape[0]

    epw = -(-e // (NW * BK)) * BK
    epad = epw * NW
    nb = epw // BK
    pad = epad - e
    src = jnp.concatenate([edge_index[0], jnp.zeros((pad,), jnp.int32)])
    dst = jnp.concatenate([edge_index[1], jnp.zeros((pad,), jnp.int32)])
    ew = jnp.concatenate([edge_weight, jnp.zeros((pad,), jnp.float32)])
    src_w = src.reshape(NW, nb, BK)
    dst_w = dst.reshape(NW, nb, BK)
    ew_w = ew.reshape(NW, nb, BK)

    h = _tc_linear(x, Win.T, bin_, relu=True)

    degp = _sc_deg(dst_w, ew_w, n, nb)
    dinv, dinv2 = _tc_dinv(degp, n)
    nrm_w = _sc_norm(src_w, dst_w, ew_w, dinv.reshape(n), n, nb)

    npad = -(-n // (NS * 8)) * (NS * 8)
    zeros = jnp.zeros((npad, d), jnp.float32)
    dinv2_col = dinv2.reshape(n, 1)
    for i in range(nlayer):
        g = _tc_linear(h, convW[i].T, None, relu=False)
        scat = _sc_agg(g, src_w, dst_w, nrm_w, zeros, n, d, nb)
        h = _tc_epilogue(scat, g, dinv2_col, convB[i], lnG[i], lnB[i], h)
    return h
